# baseline (speedup 1.0000x reference)
import functools

import jax
import jax.numpy as jnp
from jax import lax
from jax.experimental import pallas as pl
from jax.experimental.pallas import tpu as pltpu
from jax.experimental.pallas import tpu_sc as plsc

VOCAB = 1000000
PCA = 16
HIDDEN = 128
B = 16384

_info = plsc.get_sparse_core_info()
_NC, _NS = _info.num_cores, _info.num_subcores
NW = _NC * _NS
BPW = B // NW


def _gather_body(x_hbm, c_hbm, h_hbm, outc_hbm, outh_hbm,
                 idx_v, rows_c, rows_h, sem_c, sem_h):
    wid = lax.axis_index("s") * _NC + lax.axis_index("c")
    base = wid * BPW
    pltpu.sync_copy(x_hbm.at[pl.ds(base, BPW)], idx_v)
    cp_c = pltpu.async_copy(c_hbm.at[idx_v], rows_c, sem_c)
    cp_h = pltpu.async_copy(h_hbm.at[idx_v], rows_h, sem_h)
    cp_c.wait()
    pltpu.sync_copy(rows_c, outc_hbm.at[pl.ds(base, BPW)])
    cp_h.wait()
    pltpu.sync_copy(rows_h, outh_hbm.at[pl.ds(base, BPW)])


_sc_gather = functools.partial(
    pl.kernel,
    mesh=plsc.VectorSubcoreMesh(core_axis_name="c", subcore_axis_name="s"),
    out_type=[jax.ShapeDtypeStruct((B, PCA), jnp.float32),
              jax.ShapeDtypeStruct((B, PCA), jnp.float32)],
    scratch_types=[
        pltpu.VMEM((BPW,), jnp.int32),
        pltpu.VMEM((BPW, PCA), jnp.float32),
        pltpu.VMEM((BPW, PCA), jnp.float32),
        pltpu.SemaphoreType.DMA,
        pltpu.SemaphoreType.DMA,
    ],
    compiler_params=pltpu.CompilerParams(use_tc_tiling_on_sc=False),
)(_gather_body)


_BB = 2048


def _proj_body(ec_ref, eh_ref, wc_ref, wh_ref, bc_ref, bh_ref,
               oc_ref, oh_ref):
    oc_ref[...] = (
        jnp.dot(ec_ref[...], wc_ref[...], preferred_element_type=jnp.float32)
        + bc_ref[...])
    oh_ref[...] = (
        jnp.dot(eh_ref[...], wh_ref[...], preferred_element_type=jnp.float32)
        + bh_ref[...])


def _project(emb_c, emb_h, Wct, Wht, bc2, bh2):
    grid = B // _BB
    return pl.pallas_call(
        _proj_body,
        grid=(grid,),
        in_specs=[
            pl.BlockSpec((_BB, PCA), lambda i: (i, 0)),
            pl.BlockSpec((_BB, PCA), lambda i: (i, 0)),
            pl.BlockSpec((PCA, HIDDEN), lambda i: (0, 0)),
            pl.BlockSpec((PCA, HIDDEN), lambda i: (0, 0)),
            pl.BlockSpec((1, HIDDEN), lambda i: (0, 0)),
            pl.BlockSpec((1, HIDDEN), lambda i: (0, 0)),
        ],
        out_specs=[
            pl.BlockSpec((_BB, HIDDEN), lambda i: (i, 0)),
            pl.BlockSpec((_BB, HIDDEN), lambda i: (i, 0)),
        ],
        out_shape=[
            jax.ShapeDtypeStruct((B, HIDDEN), jnp.float32),
            jax.ShapeDtypeStruct((B, HIDDEN), jnp.float32),
        ],
    )(emb_c, emb_h, Wct, Wht, bc2, bh2)


def kernel(x, c_table, h_table, Wc, bc, Wh, bh):
    emb_c, emb_h = _sc_gather(x.astype(jnp.int32), c_table, h_table)
    oc, oh = _project(emb_c, emb_h, Wc.T, Wh.T,
                      bc.reshape(1, HIDDEN), bh.reshape(1, HIDDEN))
    return (oc.reshape(1, B, HIDDEN), oh.reshape(1, B, HIDDEN))
